# Initial kernel scaffold; baseline (speedup 1.0000x reference)
import jax
import jax.numpy as jnp
from jax.experimental import pallas as pl


def kernel(z_e, embedding):
    raise NotImplementedError("write your pallas kernel here")


# speedup vs baseline: 1.0944x; 1.0944x over previous
import functools

import jax
import jax.numpy as jnp
from jax import lax
from jax.experimental import pallas as pl
from jax.experimental.pallas import tpu as pltpu
from jax.experimental.pallas import tpu_sc as plsc

_K = 8192
_D = 256
_N = 16384
_BETA = 0.25

_TN = 512
_TK = 2048
_NK = _K // _TK
_NN = _N // _TN
_LOSS_SCALE = (1.0 + _BETA) / (_N * _D)

_NC = 2
_NS = 16
_NW = _NC * _NS
_RPW = _N // _NW
_GC = 128
_NCH = _RPW // _GC


def _vq_argmin_body(z_ref, et_ref, codes_ref, loss_ref, bv_ref, bi_ref):
    k = pl.program_id(0)
    n = pl.program_id(1)
    z = z_ref[...]
    et = et_ref[...]
    s = jnp.dot(z.astype(jnp.bfloat16), et.astype(jnp.bfloat16),
                preferred_element_type=jnp.float32)
    zsq = jnp.sum(z * z, axis=1, keepdims=True)
    esq = jnp.sum(et * et, axis=0, keepdims=True)
    dist = (zsq + esq) - 2.0 * s
    m = jnp.min(dist, axis=1, keepdims=True)
    col = lax.broadcasted_iota(jnp.int32, (_TN, _TK), 1) + k * _TK
    li = jnp.min(jnp.where(dist == m, col, _K), axis=1, keepdims=True)
    rows = pl.ds(n * _TN, _TN)

    @pl.when(k == 0)
    def _():
        bv_ref[rows, :] = m
        bi_ref[rows, :] = li

    @pl.when(k > 0)
    def _():
        pv = bv_ref[rows, :]
        pi = bi_ref[rows, :]
        upd = m < pv
        bv_ref[rows, :] = jnp.where(upd, m, pv)
        bi_ref[rows, :] = jnp.where(upd, li, pi)

    codes_ref[...] = bi_ref[rows, :]

    @pl.when(k == _NK - 1)
    def _():
        @pl.when(n == 0)
        def _():
            loss_ref[...] = jnp.zeros((1, 1), jnp.float32)

        loss_ref[...] = loss_ref[...] + jnp.sum(bv_ref[rows, :]).reshape(1, 1)

        @pl.when(n == _NN - 1)
        def _():
            loss_ref[...] = loss_ref[...] * _LOSS_SCALE


def _vq_argmin(z_flat, e_t):
    return pl.pallas_call(
        _vq_argmin_body,
        grid=(_NK, _NN),
        in_specs=[
            pl.BlockSpec((_TN, _D), lambda k, n: (n, 0)),
            pl.BlockSpec((_D, _TK), lambda k, n: (0, k)),
        ],
        out_specs=[
            pl.BlockSpec((_TN, 1), lambda k, n: (n, 0)),
            pl.BlockSpec((1, 1), lambda k, n: (0, 0)),
        ],
        out_shape=[
            jax.ShapeDtypeStruct((_N, 1), jnp.int32),
            jax.ShapeDtypeStruct((1, 1), jnp.float32),
        ],
        scratch_shapes=[
            pltpu.VMEM((_N, 1), jnp.float32),
            pltpu.VMEM((_N, 1), jnp.int32),
        ],
    )(z_flat, e_t)


def _sc_gather(embedding, codes):
    mesh = plsc.VectorSubcoreMesh(core_axis_name="c", subcore_axis_name="s")

    @functools.partial(
        pl.kernel,
        mesh=mesh,
        out_type=jax.ShapeDtypeStruct((_N, _D), jnp.float32),
        scratch_types=[
            pltpu.VMEM((_NCH, _GC), jnp.int32),
            pltpu.VMEM((_GC, _D), jnp.float32),
            pltpu.SemaphoreType.DMA,
        ],
    )
    def _gk(table_hbm, idx_hbm, out_hbm, idx_v, rows_v, sem):
        wid = lax.axis_index("s") * _NC + lax.axis_index("c")
        base = wid * _RPW
        for c in range(_NCH):
            pltpu.sync_copy(idx_hbm.at[pl.ds(base + c * _GC, _GC)], idx_v.at[c])
            pltpu.async_copy(table_hbm.at[idx_v.at[c]], rows_v, sem).wait()
            pltpu.sync_copy(rows_v, out_hbm.at[pl.ds(base + c * _GC, _GC)])

    return _gk(embedding, codes)


def kernel(z_e, embedding):
    B, Dc, H, W = z_e.shape
    z_flat = jnp.transpose(z_e, (0, 2, 3, 1)).reshape(-1, Dc)
    e_t = embedding.T
    codes2d, loss11 = _vq_argmin(z_flat, e_t)
    codes = codes2d.reshape(-1)
    z_q_flat = _sc_gather(embedding, codes)
    z_q = jnp.transpose(z_q_flat.reshape(B, H, W, Dc), (0, 3, 1, 2))
    return (z_q, codes.reshape(B, H, W), loss11.reshape(()))
